# baseline (speedup 1.0000x reference)
import contextlib
import functools

import numpy as np
import jax
import jax.numpy as jnp
from jax import lax
from jax.experimental import pallas as pl
from jax.experimental.pallas import tpu as pltpu
from jax.experimental.pallas import tpu_sc as plsc

NPERM = 4
L = 16
NC = 2
NS = 16
NW = NC * NS
CH = 80
NBUF = 3


@functools.lru_cache(maxsize=None)
def _perm_tables(n):
    try:
        try:
            ctx = jax.default_device(jax.local_devices(backend="cpu")[0])
        except Exception:
            ctx = contextlib.nullcontext()
        with jax.ensure_compile_time_eval(), ctx:
            perms = [
                np.asarray(
                    jax.random.permutation(
                        jax.random.fold_in(jax.random.key(42), i), n
                    )
                ).astype(np.int32)
                for i in range(NPERM)
            ]
        return np.concatenate(perms)
    except Exception:
        return None


def _perm_tables_traced(n):
    perms = [
        jax.random.permutation(
            jax.random.fold_in(jax.random.key(42), i), n
        ).astype(jnp.int32)
        for i in range(NPERM)
    ]
    return jnp.concatenate(perms)


def _sc_segment_sums(x, src, dst, tab):
    n, d = x.shape
    e = src.shape[0]
    nchunk = e // CH
    assert e % CH == 0 and n % CH == 0 and nchunk % NW == 0
    nt = nchunk // NW
    nrchunk = n // CH
    n_pad = -(-n // (NS * CH)) * (NS * CH)
    rpt = n_pad // NS

    mesh = plsc.VectorSubcoreMesh(core_axis_name="c", subcore_axis_name="s")

    out_type = (
        jax.ShapeDtypeStruct((NC * n_pad, d), jnp.float32),
        jax.ShapeDtypeStruct((NPERM * n, d), jnp.float32),
    )
    scratch = [
        pltpu.VMEM((n,), jnp.int32),
        [pltpu.VMEM((CH,), jnp.int32)] * NBUF,
        [pltpu.VMEM((CH,), jnp.int32)] * NBUF,
        [pltpu.VMEM((CH,), jnp.int32)] * NBUF,
        [pltpu.VMEM((CH,), jnp.int32)] * NBUF,
        [pltpu.VMEM((CH, d), jnp.float32)] * NBUF,
        pltpu.VMEM((CH,), jnp.int32),
        [pltpu.SemaphoreType.DMA] * NBUF,
        [pltpu.SemaphoreType.DMA] * NBUF,
        [pltpu.SemaphoreType.DMA] * NBUF,
        pltpu.VMEM_SHARED((n_pad, d), jnp.float32),
    ]

    @functools.partial(
        pl.kernel, out_type=out_type, mesh=mesh, scratch_types=scratch,
        compiler_params=pltpu.CompilerParams(needs_layout_passes=False),
    )
    def sc_kernel(x_h, src_h, dst_h, tab_h, s_h, r_h,
                  tab_v, src_v, dst_v, gidx_v, sidx_v, rows_v,
                  ridx_v, gsem, ssem, isem, acc_sh):
        cid = lax.axis_index("c")
        sid = lax.axis_index("s")
        wid = sid * NC + cid

        def zrow(r_, _):
            for j in range(d // L):
                rows_v[0][r_, pl.ds(j * L, L)] = jnp.zeros((L,), jnp.float32)
            return 0
        lax.fori_loop(0, CH, zrow, 0)

        zbase = sid * rpt
        for off in range(0, rpt, CH):
            pltpu.sync_copy(rows_v[0].at[pl.ds(0, CH)],
                            acc_sh.at[pl.ds(zbase + off, CH)])
        plsc.subcore_barrier()

        def fire_idx(t, b):
            cbase = (wid + t * NW) * CH
            pltpu.async_copy(src_h.at[pl.ds(cbase, CH)], src_v[b], isem[b])
            pltpu.async_copy(dst_h.at[pl.ds(cbase, CH)], dst_v[b], isem[b])

        def wait_idx(t, b):
            cbase = (wid + t * NW) * CH
            pltpu.make_async_copy(src_h.at[pl.ds(cbase, CH)], src_v[b],
                                  isem[b]).wait()
            pltpu.make_async_copy(dst_h.at[pl.ds(cbase, CH)], dst_v[b],
                                  isem[b]).wait()

        def translate(b):
            for kk in range(CH // L):
                sl = pl.ds(kk * L, L)
                g1 = plsc.load_gather(tab_v, [src_v[b][sl]])
                gidx_v[b][sl] = plsc.load_gather(tab_v, [g1])
                sidx_v[b][sl] = plsc.load_gather(tab_v, [dst_v[b][sl]])

        def fire_gather(b):
            pltpu.async_copy(x_h.at[gidx_v[b]], rows_v[b], gsem[b])

        def wait_gather(b):
            pltpu.make_async_copy(x_h.at[gidx_v[b]], rows_v[b],
                                  gsem[b]).wait()

        def fire_scatter(b):
            pltpu.async_copy(rows_v[b], acc_sh.at[sidx_v[b]], ssem[b],
                             add=True)

        def wait_scatter(b):
            pltpu.make_async_copy(rows_v[b], acc_sh.at[sidx_v[b]],
                                  ssem[b]).wait()

        for i in range(NPERM):
            pltpu.sync_copy(tab_h.at[pl.ds(i * n, n)], tab_v)

            for c in range(NBUF):
                fire_idx(c, c)
            for c in range(2):
                wait_idx(c, c)
                translate(c)
                fire_gather(c)

            def estep(t, m):
                pm = (m + 2) % NBUF

                @pl.when(t + NBUF < nt)
                def _():
                    fire_idx(t + NBUF, m)

                @pl.when(t >= 1)
                def _():
                    wait_scatter(pm)

                @pl.when(t + 2 < nt)
                def _():
                    wait_idx(t + 2, pm)
                    translate(pm)
                    fire_gather(pm)
                wait_gather(m)
                fire_scatter(m)

            def ebody(t, _):
                for m in range(NBUF):
                    @pl.when(t % NBUF == m)
                    def _(m=m):
                        estep(t, m)
                return 0

            lax.fori_loop(0, nt, ebody, 0)
            wait_scatter((nt - 1) % NBUF)

            nrt = (nrchunk - wid + NW - 1) // NW

            def rbody(t, _):
                rbase = (wid + t * NW) * CH
                pltpu.sync_copy(tab_h.at[pl.ds(i * n + rbase, CH)], ridx_v)
                pltpu.async_copy(x_h.at[ridx_v], rows_v[0], gsem[0]).wait()
                pltpu.sync_copy(rows_v[0], r_h.at[pl.ds(i * n + rbase, CH)])
                return 0

            lax.fori_loop(0, nrt, rbody, 0)

        plsc.subcore_barrier()

        for off in range(0, rpt, CH):
            pltpu.sync_copy(acc_sh.at[pl.ds(zbase + off, CH)],
                            s_h.at[pl.ds(cid * n_pad + zbase + off, CH)])

    return sc_kernel(x, src, dst, tab)


def _final_matmul(s2, r4, w_nbr, w_root, b2):
    n = r4.shape[0] // NPERM
    d = r4.shape[1]
    n_pad = s2.shape[0] // NC
    bm = CH
    nblk = n // bm
    s1_off = n_pad // bm
    assert n_pad % bm == 0 and n % bm == 0

    def body(s0_ref, s1_ref, r0_ref, r1_ref, r2_ref, r3_ref,
             wn_ref, wr_ref, b_ref, o_ref):
        s = s0_ref[...] + s1_ref[...]
        r = (r0_ref[...] + r1_ref[...]) + (r2_ref[...] + r3_ref[...])
        o_ref[...] = (
            jnp.dot(s, wn_ref[...], preferred_element_type=jnp.float32,
                    precision=lax.Precision.HIGHEST)
            + jnp.dot(r, wr_ref[...],
                      preferred_element_type=jnp.float32,
                      precision=lax.Precision.HIGHEST)
        ) * (1.0 / NPERM) + b_ref[...]

    r_specs = [
        pl.BlockSpec((bm, d), (lambda k: (lambda i: (i + k * nblk, 0)))(k))
        for k in range(NPERM)
    ]
    return pl.pallas_call(
        body,
        grid=(nblk,),
        in_specs=[
            pl.BlockSpec((bm, d), lambda i: (i, 0)),
            pl.BlockSpec((bm, d), lambda i: (i + s1_off, 0)),
            *r_specs,
            pl.BlockSpec((d, d), lambda i: (0, 0)),
            pl.BlockSpec((d, d), lambda i: (0, 0)),
            pl.BlockSpec((1, d), lambda i: (0, 0)),
        ],
        out_specs=pl.BlockSpec((bm, d), lambda i: (i, 0)),
        out_shape=jax.ShapeDtypeStruct((n, d), jnp.float32),
    )(s2, s2, r4, r4, r4, r4, w_nbr, w_root, b2)


def kernel(x, edge_index, W_root, W_nbr, b):
    n, d = x.shape
    tab_np = _perm_tables(n)
    tab = jnp.asarray(tab_np) if tab_np is not None else _perm_tables_traced(n)
    s2, r4 = _sc_segment_sums(x, edge_index[0], edge_index[1], tab)
    return _final_matmul(s2, r4, W_nbr, W_root, b.reshape(1, d))


# speedup vs baseline: 1.0001x; 1.0001x over previous
import contextlib
import functools

import numpy as np
import jax
import jax.numpy as jnp
from jax import lax
from jax.experimental import pallas as pl
from jax.experimental.pallas import tpu as pltpu
from jax.experimental.pallas import tpu_sc as plsc

NPERM = 4
L = 16
NC = 2
NS = 16
NW = NC * NS
CH = 80
NBUF = 3


@functools.lru_cache(maxsize=None)
def _perm_tables(n):
    try:
        try:
            ctx = jax.default_device(jax.local_devices(backend="cpu")[0])
        except Exception:
            ctx = contextlib.nullcontext()
        with jax.ensure_compile_time_eval(), ctx:
            perms = [
                np.asarray(
                    jax.random.permutation(
                        jax.random.fold_in(jax.random.key(42), i), n
                    )
                ).astype(np.int32)
                for i in range(NPERM)
            ]
        return np.concatenate(perms)
    except Exception:
        return None


def _perm_tables_traced(n):
    perms = [
        jax.random.permutation(
            jax.random.fold_in(jax.random.key(42), i), n
        ).astype(jnp.int32)
        for i in range(NPERM)
    ]
    return jnp.concatenate(perms)


def _sc_segment_sums(x, src, dst, tab):
    n, d = x.shape
    e = src.shape[0]
    nchunk = e // CH
    assert e % CH == 0 and n % CH == 0 and nchunk % NW == 0
    nt = nchunk // NW
    nrchunk = n // CH
    n_pad = -(-n // (NS * CH)) * (NS * CH)
    rpt = n_pad // NS

    mesh = plsc.VectorSubcoreMesh(core_axis_name="c", subcore_axis_name="s")

    out_type = (
        jax.ShapeDtypeStruct((NC * n_pad, d), jnp.float32),
        jax.ShapeDtypeStruct((NPERM * n, d), jnp.float32),
    )
    scratch = [
        pltpu.VMEM((n,), jnp.int32),
        [pltpu.VMEM((CH,), jnp.int32)] * NBUF,
        [pltpu.VMEM((CH,), jnp.int32)] * NBUF,
        [pltpu.VMEM((CH,), jnp.int32)] * NBUF,
        [pltpu.VMEM((CH,), jnp.int32)] * NBUF,
        [pltpu.VMEM((CH, d), jnp.float32)] * NBUF,
        pltpu.VMEM((CH,), jnp.int32),
        [pltpu.SemaphoreType.DMA] * NBUF,
        [pltpu.SemaphoreType.DMA] * NBUF,
        [pltpu.SemaphoreType.DMA] * NBUF,
        pltpu.VMEM_SHARED((n_pad, d), jnp.float32),
    ]

    @functools.partial(
        pl.kernel, out_type=out_type, mesh=mesh, scratch_types=scratch,
        compiler_params=pltpu.CompilerParams(needs_layout_passes=False),
    )
    def sc_kernel(x_h, src_h, dst_h, tab_h, s_h, r_h,
                  tab_v, src_v, dst_v, gidx_v, sidx_v, rows_v,
                  ridx_v, gsem, ssem, isem, acc_sh):
        cid = lax.axis_index("c")
        sid = lax.axis_index("s")
        wid = sid * NC + cid

        def zrow(r_, _):
            for j in range(d // L):
                rows_v[0][r_, pl.ds(j * L, L)] = jnp.zeros((L,), jnp.float32)
            return 0
        lax.fori_loop(0, CH, zrow, 0)

        zbase = sid * rpt
        for off in range(0, rpt, CH):
            pltpu.sync_copy(rows_v[0].at[pl.ds(0, CH)],
                            acc_sh.at[pl.ds(zbase + off, CH)])
        plsc.subcore_barrier()

        def fire_idx(t, b):
            cbase = (wid + t * NW) * CH
            pltpu.async_copy(src_h.at[pl.ds(cbase, CH)], src_v[b], isem[b])
            pltpu.async_copy(dst_h.at[pl.ds(cbase, CH)], dst_v[b], isem[b])

        def wait_idx(t, b):
            cbase = (wid + t * NW) * CH
            pltpu.make_async_copy(src_h.at[pl.ds(cbase, CH)], src_v[b],
                                  isem[b]).wait()
            pltpu.make_async_copy(dst_h.at[pl.ds(cbase, CH)], dst_v[b],
                                  isem[b]).wait()

        def translate(b):
            for kk in range(CH // L):
                sl = pl.ds(kk * L, L)
                g1 = plsc.load_gather(tab_v, [src_v[b][sl]])
                gidx_v[b][sl] = plsc.load_gather(tab_v, [g1])
                sidx_v[b][sl] = plsc.load_gather(tab_v, [dst_v[b][sl]])

        h = CH // 2

        def fire_gather(b):
            pltpu.async_copy(x_h.at[gidx_v[b].at[pl.ds(0, h)]],
                             rows_v[b].at[pl.ds(0, h)], gsem[b])
            pltpu.async_copy(x_h.at[gidx_v[b].at[pl.ds(h, h)]],
                             rows_v[b].at[pl.ds(h, h)], gsem[b])

        def wait_gather(b):
            pltpu.make_async_copy(x_h.at[gidx_v[b].at[pl.ds(0, h)]],
                                  rows_v[b].at[pl.ds(0, h)], gsem[b]).wait()
            pltpu.make_async_copy(x_h.at[gidx_v[b].at[pl.ds(h, h)]],
                                  rows_v[b].at[pl.ds(h, h)], gsem[b]).wait()

        def fire_scatter(b):
            pltpu.async_copy(rows_v[b], acc_sh.at[sidx_v[b]], ssem[b],
                             add=True)

        def wait_scatter(b):
            pltpu.make_async_copy(rows_v[b], acc_sh.at[sidx_v[b]],
                                  ssem[b]).wait()

        for i in range(NPERM):
            pltpu.sync_copy(tab_h.at[pl.ds(i * n, n)], tab_v)

            for c in range(NBUF):
                fire_idx(c, c)
            for c in range(2):
                wait_idx(c, c)
                translate(c)
                fire_gather(c)

            def estep(t, m):
                pm = (m + 2) % NBUF

                @pl.when(t + NBUF < nt)
                def _():
                    fire_idx(t + NBUF, m)

                @pl.when(t >= 1)
                def _():
                    wait_scatter(pm)

                @pl.when(t + 2 < nt)
                def _():
                    wait_idx(t + 2, pm)
                    translate(pm)
                    fire_gather(pm)
                wait_gather(m)
                fire_scatter(m)

            def ebody(t, _):
                for m in range(NBUF):
                    @pl.when(t % NBUF == m)
                    def _(m=m):
                        estep(t, m)
                return 0

            lax.fori_loop(0, nt, ebody, 0)
            wait_scatter((nt - 1) % NBUF)

            nrt = (nrchunk - wid + NW - 1) // NW

            def rbody(t, _):
                rbase = (wid + t * NW) * CH
                pltpu.sync_copy(tab_h.at[pl.ds(i * n + rbase, CH)], ridx_v)
                pltpu.async_copy(x_h.at[ridx_v], rows_v[0], gsem[0]).wait()
                pltpu.sync_copy(rows_v[0], r_h.at[pl.ds(i * n + rbase, CH)])
                return 0

            lax.fori_loop(0, nrt, rbody, 0)

        plsc.subcore_barrier()

        for off in range(0, rpt, CH):
            pltpu.sync_copy(acc_sh.at[pl.ds(zbase + off, CH)],
                            s_h.at[pl.ds(cid * n_pad + zbase + off, CH)])

    return sc_kernel(x, src, dst, tab)


def _final_matmul(s2, r4, w_nbr, w_root, b2):
    n = r4.shape[0] // NPERM
    d = r4.shape[1]
    n_pad = s2.shape[0] // NC
    bm = CH
    nblk = n // bm
    s1_off = n_pad // bm
    assert n_pad % bm == 0 and n % bm == 0

    def body(s0_ref, s1_ref, r0_ref, r1_ref, r2_ref, r3_ref,
             wn_ref, wr_ref, b_ref, o_ref):
        s = s0_ref[...] + s1_ref[...]
        r = (r0_ref[...] + r1_ref[...]) + (r2_ref[...] + r3_ref[...])
        o_ref[...] = (
            jnp.dot(s, wn_ref[...], preferred_element_type=jnp.float32,
                    precision=lax.Precision.HIGHEST)
            + jnp.dot(r, wr_ref[...],
                      preferred_element_type=jnp.float32,
                      precision=lax.Precision.HIGHEST)
        ) * (1.0 / NPERM) + b_ref[...]

    r_specs = [
        pl.BlockSpec((bm, d), (lambda k: (lambda i: (i + k * nblk, 0)))(k))
        for k in range(NPERM)
    ]
    return pl.pallas_call(
        body,
        grid=(nblk,),
        in_specs=[
            pl.BlockSpec((bm, d), lambda i: (i, 0)),
            pl.BlockSpec((bm, d), lambda i: (i + s1_off, 0)),
            *r_specs,
            pl.BlockSpec((d, d), lambda i: (0, 0)),
            pl.BlockSpec((d, d), lambda i: (0, 0)),
            pl.BlockSpec((1, d), lambda i: (0, 0)),
        ],
        out_specs=pl.BlockSpec((bm, d), lambda i: (i, 0)),
        out_shape=jax.ShapeDtypeStruct((n, d), jnp.float32),
    )(s2, s2, r4, r4, r4, r4, w_nbr, w_root, b2)


def kernel(x, edge_index, W_root, W_nbr, b):
    n, d = x.shape
    tab_np = _perm_tables(n)
    tab = jnp.asarray(tab_np) if tab_np is not None else _perm_tables_traced(n)
    s2, r4 = _sc_segment_sums(x, edge_index[0], edge_index[1], tab)
    return _final_matmul(s2, r4, W_nbr, W_root, b.reshape(1, d))


# speedup vs baseline: 1.0303x; 1.0302x over previous
import contextlib
import functools

import numpy as np
import jax
import jax.numpy as jnp
from jax import lax
from jax.experimental import pallas as pl
from jax.experimental.pallas import tpu as pltpu
from jax.experimental.pallas import tpu_sc as plsc

NPERM = 4
L = 16
NC = 2
NS = 16
NW = NC * NS
CH = 80
NBUF = 3


@functools.lru_cache(maxsize=None)
def _perm_tables(n):
    try:
        try:
            ctx = jax.default_device(jax.local_devices(backend="cpu")[0])
        except Exception:
            ctx = contextlib.nullcontext()
        with jax.ensure_compile_time_eval(), ctx:
            perms = [
                np.asarray(
                    jax.random.permutation(
                        jax.random.fold_in(jax.random.key(42), i), n
                    )
                ).astype(np.int32)
                for i in range(NPERM)
            ]
        return np.concatenate(perms)
    except Exception:
        return None


def _perm_tables_traced(n):
    perms = [
        jax.random.permutation(
            jax.random.fold_in(jax.random.key(42), i), n
        ).astype(jnp.int32)
        for i in range(NPERM)
    ]
    return jnp.concatenate(perms)


def _sc_segment_sums(x, src, dst, tab):
    n, d = x.shape
    e = src.shape[0]
    nchunk = e // CH
    assert e % CH == 0 and n % CH == 0 and nchunk % NW == 0
    nt = nchunk // NW
    nrchunk = n // CH
    n_pad = -(-n // (NS * CH)) * (NS * CH)
    rpt = n_pad // NS

    mesh = plsc.VectorSubcoreMesh(core_axis_name="c", subcore_axis_name="s")

    out_type = (
        jax.ShapeDtypeStruct((NC * n_pad, d), jnp.float32),
        jax.ShapeDtypeStruct((NPERM * n, d), jnp.float32),
    )
    scratch = [
        pltpu.VMEM((n,), jnp.int32),
        [pltpu.VMEM((CH,), jnp.int32)] * NBUF,
        [pltpu.VMEM((CH,), jnp.int32)] * NBUF,
        [pltpu.VMEM((CH,), jnp.int32)] * NBUF,
        [pltpu.VMEM((CH,), jnp.int32)] * NBUF,
        [pltpu.VMEM((CH, d), jnp.float32)] * NBUF,
        [pltpu.VMEM((CH,), jnp.int32)] * 2,
        [pltpu.SemaphoreType.DMA] * NBUF,
        [pltpu.SemaphoreType.DMA] * NBUF,
        [pltpu.SemaphoreType.DMA] * NBUF,
        pltpu.VMEM_SHARED((n_pad, d), jnp.float32),
    ]

    @functools.partial(
        pl.kernel, out_type=out_type, mesh=mesh, scratch_types=scratch,
        compiler_params=pltpu.CompilerParams(needs_layout_passes=False),
    )
    def sc_kernel(x_h, src_h, dst_h, tab_h, s_h, r_h,
                  tab_v, src_v, dst_v, gidx_v, sidx_v, rows_v,
                  ridx_v, gsem, ssem, isem, acc_sh):
        cid = lax.axis_index("c")
        sid = lax.axis_index("s")
        wid = sid * NC + cid

        def zrow(r_, _):
            for j in range(d // L):
                rows_v[0][r_, pl.ds(j * L, L)] = jnp.zeros((L,), jnp.float32)
            return 0
        lax.fori_loop(0, CH, zrow, 0)

        zbase = sid * rpt
        for off in range(0, rpt, CH):
            pltpu.sync_copy(rows_v[0].at[pl.ds(0, CH)],
                            acc_sh.at[pl.ds(zbase + off, CH)])
        plsc.subcore_barrier()

        def fire_idx(t, b):
            cbase = (wid + t * NW) * CH
            pltpu.async_copy(src_h.at[pl.ds(cbase, CH)], src_v[b], isem[b])
            pltpu.async_copy(dst_h.at[pl.ds(cbase, CH)], dst_v[b], isem[b])

        def wait_idx(t, b):
            cbase = (wid + t * NW) * CH
            pltpu.make_async_copy(src_h.at[pl.ds(cbase, CH)], src_v[b],
                                  isem[b]).wait()
            pltpu.make_async_copy(dst_h.at[pl.ds(cbase, CH)], dst_v[b],
                                  isem[b]).wait()

        def translate(b):
            for kk in range(CH // L):
                sl = pl.ds(kk * L, L)
                g1 = plsc.load_gather(tab_v, [src_v[b][sl]])
                gidx_v[b][sl] = plsc.load_gather(tab_v, [g1])
                sidx_v[b][sl] = plsc.load_gather(tab_v, [dst_v[b][sl]])

        def fire_gather(b):
            pltpu.async_copy(x_h.at[gidx_v[b]], rows_v[b], gsem[b])

        def wait_gather(b):
            pltpu.make_async_copy(x_h.at[gidx_v[b]], rows_v[b],
                                  gsem[b]).wait()

        def fire_scatter(b):
            pltpu.async_copy(rows_v[b], acc_sh.at[sidx_v[b]], ssem[b],
                             add=True)

        def wait_scatter(b):
            pltpu.make_async_copy(rows_v[b], acc_sh.at[sidx_v[b]],
                                  ssem[b]).wait()

        for i in range(NPERM):
            pltpu.sync_copy(tab_h.at[pl.ds(i * n, n)], tab_v)

            for c in range(NBUF):
                fire_idx(c, c)
            for c in range(2):
                wait_idx(c, c)
                translate(c)
                fire_gather(c)

            def estep(t, m):
                pm = (m + 2) % NBUF

                @pl.when(t + NBUF < nt)
                def _():
                    fire_idx(t + NBUF, m)

                @pl.when(t >= 1)
                def _():
                    wait_scatter(pm)

                @pl.when(t + 2 < nt)
                def _():
                    wait_idx(t + 2, pm)
                    translate(pm)
                    fire_gather(pm)
                wait_gather(m)
                fire_scatter(m)

            def ebody(t, _):
                for m in range(NBUF):
                    @pl.when(t % NBUF == m)
                    def _(m=m):
                        estep(t, m)
                return 0

            lax.fori_loop(0, nt, ebody, 0)
            wait_scatter((nt - 1) % NBUF)

            nrt = (nrchunk - wid + NW - 1) // NW

            def rcopy(t, b):
                rbase = (wid + t * NW) * CH
                for kk in range(CH // L):
                    ridx_v[b][pl.ds(kk * L, L)] = tab_v[
                        pl.ds(rbase + kk * L, L)]

            def fire_rg(b):
                pltpu.async_copy(x_h.at[ridx_v[b]], rows_v[b], gsem[b])

            def wait_rg(b):
                pltpu.make_async_copy(x_h.at[ridx_v[b]], rows_v[b],
                                      gsem[b]).wait()

            rcopy(0, 0)
            fire_rg(0)

            def rstep(t, m):
                nb = 1 - m

                @pl.when(t + 1 < nrt)
                def _():
                    rcopy(t + 1, nb)
                    fire_rg(nb)
                wait_rg(m)
                rbase = (wid + t * NW) * CH
                pltpu.sync_copy(rows_v[m],
                                r_h.at[pl.ds(i * n + rbase, CH)])

            def rbody(t, _):
                @pl.when(t % 2 == 0)
                def _():
                    rstep(t, 0)

                @pl.when(t % 2 == 1)
                def _():
                    rstep(t, 1)
                return 0

            lax.fori_loop(0, nrt, rbody, 0)

        plsc.subcore_barrier()

        for off in range(0, rpt, CH):
            pltpu.sync_copy(acc_sh.at[pl.ds(zbase + off, CH)],
                            s_h.at[pl.ds(cid * n_pad + zbase + off, CH)])

    return sc_kernel(x, src, dst, tab)


def _final_matmul(s2, r4, w_nbr, w_root, b2):
    n = r4.shape[0] // NPERM
    d = r4.shape[1]
    n_pad = s2.shape[0] // NC
    bm = CH
    nblk = n // bm
    s1_off = n_pad // bm
    assert n_pad % bm == 0 and n % bm == 0

    def body(s0_ref, s1_ref, r0_ref, r1_ref, r2_ref, r3_ref,
             wn_ref, wr_ref, b_ref, o_ref):
        s = s0_ref[...] + s1_ref[...]
        r = (r0_ref[...] + r1_ref[...]) + (r2_ref[...] + r3_ref[...])
        o_ref[...] = (
            jnp.dot(s, wn_ref[...], preferred_element_type=jnp.float32,
                    precision=lax.Precision.HIGHEST)
            + jnp.dot(r, wr_ref[...],
                      preferred_element_type=jnp.float32,
                      precision=lax.Precision.HIGHEST)
        ) * (1.0 / NPERM) + b_ref[...]

    r_specs = [
        pl.BlockSpec((bm, d), (lambda k: (lambda i: (i + k * nblk, 0)))(k))
        for k in range(NPERM)
    ]
    return pl.pallas_call(
        body,
        grid=(nblk,),
        in_specs=[
            pl.BlockSpec((bm, d), lambda i: (i, 0)),
            pl.BlockSpec((bm, d), lambda i: (i + s1_off, 0)),
            *r_specs,
            pl.BlockSpec((d, d), lambda i: (0, 0)),
            pl.BlockSpec((d, d), lambda i: (0, 0)),
            pl.BlockSpec((1, d), lambda i: (0, 0)),
        ],
        out_specs=pl.BlockSpec((bm, d), lambda i: (i, 0)),
        out_shape=jax.ShapeDtypeStruct((n, d), jnp.float32),
    )(s2, s2, r4, r4, r4, r4, w_nbr, w_root, b2)


def kernel(x, edge_index, W_root, W_nbr, b):
    n, d = x.shape
    tab_np = _perm_tables(n)
    tab = jnp.asarray(tab_np) if tab_np is not None else _perm_tables_traced(n)
    s2, r4 = _sc_segment_sums(x, edge_index[0], edge_index[1], tab)
    return _final_matmul(s2, r4, W_nbr, W_root, b.reshape(1, d))


# speedup vs baseline: 1.1767x; 1.1421x over previous
import contextlib
import functools

import numpy as np
import jax
import jax.numpy as jnp
from jax import lax
from jax.experimental import pallas as pl
from jax.experimental.pallas import tpu as pltpu
from jax.experimental.pallas import tpu_sc as plsc

NPERM = 4
L = 16
NC = 2
NS = 16
NW = NC * NS
CH = 80
NBUF = 3


@functools.lru_cache(maxsize=None)
def _perm_tables(n):
    try:
        try:
            ctx = jax.default_device(jax.local_devices(backend="cpu")[0])
        except Exception:
            ctx = contextlib.nullcontext()
        with jax.ensure_compile_time_eval(), ctx:
            perms = [
                np.asarray(
                    jax.random.permutation(
                        jax.random.fold_in(jax.random.key(42), i), n
                    )
                ).astype(np.int32)
                for i in range(NPERM)
            ]
        return np.concatenate(perms)
    except Exception:
        return None


def _perm_tables_traced(n):
    perms = [
        jax.random.permutation(
            jax.random.fold_in(jax.random.key(42), i), n
        ).astype(jnp.int32)
        for i in range(NPERM)
    ]
    return jnp.concatenate(perms)


def _sc_segment_sums(x, src, dst, tab):
    n, d = x.shape
    e = src.shape[0]
    nchunk = e // CH
    assert e % CH == 0 and n % CH == 0 and nchunk % NW == 0
    nt = nchunk // NW
    nrchunk = n // CH
    n_pad = -(-n // (NS * CH)) * (NS * CH)
    rpt = n_pad // NS

    mesh = plsc.VectorSubcoreMesh(core_axis_name="c", subcore_axis_name="s")

    out_type = (
        jax.ShapeDtypeStruct((NC, n_pad, d), jnp.float32),
        jax.ShapeDtypeStruct((NPERM * n, d), jnp.float32),
    )
    scratch = [
        pltpu.VMEM((n,), jnp.int32),
        [pltpu.VMEM((CH,), jnp.int32)] * NBUF,
        [pltpu.VMEM((CH,), jnp.int32)] * NBUF,
        [pltpu.VMEM((CH,), jnp.int32)] * NBUF,
        [pltpu.VMEM((CH,), jnp.int32)] * NBUF,
        [pltpu.VMEM((CH, d), jnp.float32)] * NBUF,
        [pltpu.VMEM((CH,), jnp.int32)] * 2,
        [pltpu.SemaphoreType.DMA] * NBUF,
        [pltpu.SemaphoreType.DMA] * NBUF,
        [pltpu.SemaphoreType.DMA] * NBUF,
        pltpu.VMEM_SHARED((n_pad, d), jnp.float32),
    ]

    @functools.partial(
        pl.kernel, out_type=out_type, mesh=mesh, scratch_types=scratch,
        compiler_params=pltpu.CompilerParams(needs_layout_passes=False),
    )
    def sc_kernel(x_h, src_h, dst_h, tab_h, s_h, r_h,
                  tab_v, src_v, dst_v, gidx_v, sidx_v, rows_v,
                  ridx_v, gsem, ssem, isem, acc_sh):
        cid = lax.axis_index("c")
        sid = lax.axis_index("s")
        wid = sid * NC + cid

        def zrow(r_, _):
            for j in range(d // L):
                rows_v[0][r_, pl.ds(j * L, L)] = jnp.zeros((L,), jnp.float32)
            return 0
        lax.fori_loop(0, CH, zrow, 0)

        zbase = sid * rpt
        for off in range(0, rpt, CH):
            pltpu.sync_copy(rows_v[0].at[pl.ds(0, CH)],
                            acc_sh.at[pl.ds(zbase + off, CH)])
        plsc.subcore_barrier()

        def fire_idx(t, b):
            cbase = (wid + t * NW) * CH
            pltpu.async_copy(src_h.at[pl.ds(cbase, CH)], src_v[b], isem[b])
            pltpu.async_copy(dst_h.at[pl.ds(cbase, CH)], dst_v[b], isem[b])

        def wait_idx(t, b):
            cbase = (wid + t * NW) * CH
            pltpu.make_async_copy(src_h.at[pl.ds(cbase, CH)], src_v[b],
                                  isem[b]).wait()
            pltpu.make_async_copy(dst_h.at[pl.ds(cbase, CH)], dst_v[b],
                                  isem[b]).wait()

        def translate(b):
            for kk in range(CH // L):
                sl = pl.ds(kk * L, L)
                g1 = plsc.load_gather(tab_v, [src_v[b][sl]])
                gidx_v[b][sl] = plsc.load_gather(tab_v, [g1])
                sidx_v[b][sl] = plsc.load_gather(tab_v, [dst_v[b][sl]])

        def fire_gather(b):
            pltpu.async_copy(x_h.at[gidx_v[b]], rows_v[b], gsem[b])

        def wait_gather(b):
            pltpu.make_async_copy(x_h.at[gidx_v[b]], rows_v[b],
                                  gsem[b]).wait()

        def fire_scatter(b):
            pltpu.async_copy(rows_v[b], acc_sh.at[sidx_v[b]], ssem[b],
                             add=True)

        def wait_scatter(b):
            pltpu.make_async_copy(rows_v[b], acc_sh.at[sidx_v[b]],
                                  ssem[b]).wait()

        for i in range(NPERM):
            pltpu.sync_copy(tab_h.at[pl.ds(i * n, n)], tab_v)

            for c in range(NBUF):
                fire_idx(c, c)
            for c in range(2):
                wait_idx(c, c)
                translate(c)
                fire_gather(c)

            def estep(t, m, first, fire, trans):
                pm = (m + 2) % NBUF
                if fire:
                    fire_idx(t + NBUF, m)
                if not first:
                    wait_scatter(pm)
                if trans:
                    wait_idx(t + 2, pm)
                    translate(pm)
                    fire_gather(pm)
                wait_gather(m)
                fire_scatter(m)

            lo = min(2, nt)
            hi = max(lo, nt - NBUF)
            nfull = (hi - lo) // NBUF
            for t in range(lo):
                estep(t, t % NBUF, t == 0, t + NBUF < nt, t + 2 < nt)

            def ebody(k, _):
                tb = lo + k * NBUF
                for j in range(NBUF):
                    estep(tb + j, (lo + j) % NBUF, False, True, True)
                return 0

            lax.fori_loop(0, nfull, ebody, 0)
            for t in range(lo + nfull * NBUF, nt):
                estep(t, t % NBUF, t == 0, t + NBUF < nt, t + 2 < nt)
            wait_scatter((nt - 1) % NBUF)

            nrt = (nrchunk - wid + NW - 1) // NW

            def rcopy(t, b):
                rbase = (wid + t * NW) * CH
                for kk in range(CH // L):
                    ridx_v[b][pl.ds(kk * L, L)] = tab_v[
                        pl.ds(rbase + kk * L, L)]

            def fire_rg(b):
                pltpu.async_copy(x_h.at[ridx_v[b]], rows_v[b], gsem[b])

            def wait_rg(b):
                pltpu.make_async_copy(x_h.at[ridx_v[b]], rows_v[b],
                                      gsem[b]).wait()

            rcopy(0, 0)
            fire_rg(0)

            def rstep(t, m):
                nb = 1 - m

                @pl.when(t + 1 < nrt)
                def _():
                    rcopy(t + 1, nb)
                    fire_rg(nb)
                wait_rg(m)
                rbase = (wid + t * NW) * CH
                pltpu.sync_copy(rows_v[m],
                                r_h.at[pl.ds(i * n + rbase, CH)])

            def rbody(t, _):
                @pl.when(t % 2 == 0)
                def _():
                    rstep(t, 0)

                @pl.when(t % 2 == 1)
                def _():
                    rstep(t, 1)
                return 0

            lax.fori_loop(0, nrt, rbody, 0)

        plsc.subcore_barrier()

        for off in range(0, rpt, CH):
            pltpu.sync_copy(acc_sh.at[pl.ds(zbase + off, CH)],
                            s_h.at[cid, pl.ds(zbase + off, CH)])

    return sc_kernel(x, src, dst, tab)


def _final_matmul(s3, r4, w_nbr, w_root, b2):
    n = r4.shape[0] // NPERM
    d = r4.shape[1]
    bm = 1000
    nblk = n // bm
    rblk = n // bm
    assert n % bm == 0

    def body(s0_ref, s1_ref, r0_ref, r1_ref, r2_ref, r3_ref,
             wn_ref, wr_ref, b_ref, o_ref):
        s = s0_ref[0] + s1_ref[0]
        r = (r0_ref[...] + r1_ref[...]) + (r2_ref[...] + r3_ref[...])
        o_ref[...] = (
            jnp.dot(s, wn_ref[...], preferred_element_type=jnp.float32,
                    precision=lax.Precision.HIGHEST)
            + jnp.dot(r, wr_ref[...],
                      preferred_element_type=jnp.float32,
                      precision=lax.Precision.HIGHEST)
        ) * (1.0 / NPERM) + b_ref[...]

    r_specs = [
        pl.BlockSpec((bm, d), (lambda k: (lambda i: (i + k * rblk, 0)))(k))
        for k in range(NPERM)
    ]
    return pl.pallas_call(
        body,
        grid=(nblk,),
        in_specs=[
            pl.BlockSpec((1, bm, d), lambda i: (0, i, 0)),
            pl.BlockSpec((1, bm, d), lambda i: (1, i, 0)),
            *r_specs,
            pl.BlockSpec((d, d), lambda i: (0, 0)),
            pl.BlockSpec((d, d), lambda i: (0, 0)),
            pl.BlockSpec((1, d), lambda i: (0, 0)),
        ],
        out_specs=pl.BlockSpec((bm, d), lambda i: (i, 0)),
        out_shape=jax.ShapeDtypeStruct((n, d), jnp.float32),
    )(s3, s3, r4, r4, r4, r4, w_nbr, w_root, b2)


def kernel(x, edge_index, W_root, W_nbr, b):
    n, d = x.shape
    tab_np = _perm_tables(n)
    tab = jnp.asarray(tab_np) if tab_np is not None else _perm_tables_traced(n)
    s2, r4 = _sc_segment_sums(x, edge_index[0], edge_index[1], tab)
    return _final_matmul(s2, r4, W_nbr, W_root, b.reshape(1, d))
